# baseline (speedup 1.0000x reference)
import functools

import jax
import jax.numpy as jnp
from jax import lax
from jax.experimental import pallas as pl
from jax.experimental.pallas import tpu as pltpu
from jax.experimental.pallas import tpu_sc as plsc

E = 8
D = 768
H = 128
N = 16384
BLK = 256
NB = N // BLK + E
NPAD = NB * BLK

RG = 128
RC = N // RG


def _route_body(lab_ref, dest_ref, blkexp_ref, limit_ref):
    lab = lab_ref[...]
    io0 = lax.broadcasted_iota(jnp.int32, (RC, RC), 0)
    io1 = lax.broadcasted_iota(jnp.int32, (RC, RC), 1)
    m_cols = (io0 < io1).astype(jnp.float32)
    l_rows = (io1 < io0).astype(jnp.float32)

    counts = []
    offs = []
    dest = jnp.zeros((RG, RC), dtype=jnp.int32)
    off = jnp.int32(0)
    for e in range(E):
        mask = (lab == e)
        maskf = mask.astype(jnp.float32)
        within = jnp.dot(maskf, m_cols, preferred_element_type=jnp.float32)
        rowcnt = jnp.sum(maskf, axis=1, keepdims=True)
        rowpre = jnp.dot(l_rows, rowcnt,
                         preferred_element_type=jnp.float32)
        rank = (within + rowpre).astype(jnp.int32)
        cnt = jnp.sum(mask.astype(jnp.int32))
        counts.append(cnt)
        offs.append(off)
        dest = dest + jnp.where(mask, off + rank, 0)
        padded = ((cnt + BLK - 1) // BLK) * BLK
        off = off + padded
    dest_ref[...] = dest

    brow = lax.broadcasted_iota(jnp.int32, (1, NB), 1) * BLK
    blkexp = jnp.zeros((1, NB), dtype=jnp.int32)
    limit = jnp.zeros((1, NB), dtype=jnp.int32)
    for e in range(E):
        lo = offs[e]
        hi = offs[e + 1] if e + 1 < E else off
        ind = (brow >= lo) & (brow < hi)
        blkexp = blkexp + jnp.where(ind, e, 0)
        limit = limit + jnp.where(ind, lo + counts[e], 0)
    blkexp_ref[...] = blkexp
    limit_ref[...] = limit


def _route(label2d):
    return pl.pallas_call(
        _route_body,
        out_shape=(
            jax.ShapeDtypeStruct((RG, RC), jnp.int32),
            jax.ShapeDtypeStruct((1, NB), jnp.int32),
            jax.ShapeDtypeStruct((1, NB), jnp.int32),
        ),
    )(label2d)


_SC_CH = 64


def _sc_scatter(img, dest3d):
    mesh = plsc.VectorSubcoreMesh(core_axis_name="c", subcore_axis_name="s")
    nw = mesh.num_cores * mesh.num_subcores
    tpw = N // nw
    nch = tpw // _SC_CH

    @functools.partial(
        pl.kernel,
        out_type=jax.ShapeDtypeStruct((NPAD, D), jnp.float32),
        mesh=mesh,
        scratch_types=[
            pltpu.VMEM((nch, _SC_CH), jnp.int32),
            pltpu.VMEM((2, _SC_CH, D), jnp.float32),
            pltpu.SemaphoreType.DMA,
            pltpu.SemaphoreType.DMA,
        ],
    )
    def k(img_hbm, dest_hbm, xpad_hbm, idx_v, rows_v, sem_r, sem_w):
        wid = lax.axis_index("s") * mesh.num_cores + lax.axis_index("c")
        base = wid * tpw
        pltpu.sync_copy(dest_hbm.at[wid], idx_v)

        def read(j, s):
            return pltpu.async_copy(
                img_hbm.at[pl.ds(base + j * _SC_CH, _SC_CH)],
                rows_v.at[s], sem_r)

        def scat(j, s):
            return pltpu.async_copy(rows_v.at[s], xpad_hbm.at[idx_v.at[j]],
                                    sem_w)

        rd = read(0, 0)
        wr_prev = None
        for j in range(nch):
            s = j % 2
            rd.wait()
            wr = scat(j, s)
            if j + 1 < nch:
                if wr_prev is not None:
                    wr_prev.wait()
                rd = read(j + 1, (j + 1) % 2)
            wr_prev, wr = wr, None
        wr_prev.wait()

    return k(img, dest3d)


def _sc_gather(ypad, dest3d):
    mesh = plsc.VectorSubcoreMesh(core_axis_name="c", subcore_axis_name="s")
    nw = mesh.num_cores * mesh.num_subcores
    tpw = N // nw
    nch = tpw // _SC_CH

    @functools.partial(
        pl.kernel,
        out_type=jax.ShapeDtypeStruct((N, D), jnp.float32),
        mesh=mesh,
        scratch_types=[
            pltpu.VMEM((nch, _SC_CH), jnp.int32),
            pltpu.VMEM((2, _SC_CH, D), jnp.float32),
            pltpu.SemaphoreType.DMA,
            pltpu.SemaphoreType.DMA,
        ],
    )
    def k(ypad_hbm, dest_hbm, out_hbm, idx_v, rows_v, sem_r, sem_w):
        wid = lax.axis_index("s") * mesh.num_cores + lax.axis_index("c")
        base = wid * tpw
        pltpu.sync_copy(dest_hbm.at[wid], idx_v)

        def gath(j, s):
            return pltpu.async_copy(ypad_hbm.at[idx_v.at[j]], rows_v.at[s],
                                    sem_r)

        def write(j, s):
            return pltpu.async_copy(
                rows_v.at[s],
                out_hbm.at[pl.ds(base + j * _SC_CH, _SC_CH)], sem_w)

        rd = gath(0, 0)
        wr_prev = None
        for j in range(nch):
            s = j % 2
            rd.wait()
            wr = write(j, s)
            if j + 1 < nch:
                if wr_prev is not None:
                    wr_prev.wait()
                rd = gath(j + 1, (j + 1) % 2)
            wr_prev, wr = wr, None
        wr_prev.wait()

    return k(ypad, dest3d)


def _expert_body(be_idx_ref, lim_ref, x_ref, we_ref, bee_ref, wd_ref, bd_ref,
                 y_ref, loss_ref, acc_ref):
    b = pl.program_id(0)
    x = x_ref[...]
    h = jnp.dot(x.astype(jnp.bfloat16), we_ref[0].astype(jnp.bfloat16),
                preferred_element_type=jnp.float32)
    h = h + bee_ref[0]
    y = jnp.dot(h.astype(jnp.bfloat16), wd_ref[0].astype(jnp.bfloat16),
                preferred_element_type=jnp.float32)
    y = y + bd_ref[0]
    y_ref[...] = y

    limit = lim_ref[0, b]
    row = b * BLK + lax.broadcasted_iota(jnp.int32, (BLK, 1), 0)
    diff = y - x
    sq = jnp.where(row < limit, diff * diff, 0.0)

    @pl.when(b == 0)
    def _():
        acc_ref[0] = 0.0

    acc_ref[0] += jnp.sum(sq)

    @pl.when(b == NB - 1)
    def _():
        loss_ref[...] = jnp.reshape(acc_ref[0] / (N * D), (1, 1))


def _experts(xpad, We, be, Wd, bd, blkexp, limit):
    grid_spec = pltpu.PrefetchScalarGridSpec(
        num_scalar_prefetch=2,
        grid=(NB,),
        in_specs=[
            pl.BlockSpec((BLK, D), lambda b, bexp, lim: (b, 0)),
            pl.BlockSpec((1, D, H), lambda b, bexp, lim: (bexp[0, b], 0, 0)),
            pl.BlockSpec((1, 1, H), lambda b, bexp, lim: (bexp[0, b], 0, 0)),
            pl.BlockSpec((1, H, D), lambda b, bexp, lim: (bexp[0, b], 0, 0)),
            pl.BlockSpec((1, 1, D), lambda b, bexp, lim: (bexp[0, b], 0, 0)),
        ],
        out_specs=[
            pl.BlockSpec((BLK, D), lambda b, bexp, lim: (b, 0)),
            pl.BlockSpec((1, 1), lambda b, bexp, lim: (0, 0)),
        ],
        scratch_shapes=[pltpu.SMEM((1,), jnp.float32)],
    )
    return pl.pallas_call(
        _expert_body,
        grid_spec=grid_spec,
        out_shape=(
            jax.ShapeDtypeStruct((NPAD, D), jnp.float32),
            jax.ShapeDtypeStruct((1, 1), jnp.float32),
        ),
    )(blkexp, limit, xpad, We, be.reshape(E, 1, H), Wd, bd.reshape(E, 1, D))


def kernel(img, label, We, be, Wd, bd):
    label2d = label.astype(jnp.int32).reshape(RG, RC)
    dest, blkexp, limit = _route(label2d)
    mesh = plsc.VectorSubcoreMesh(core_axis_name="c", subcore_axis_name="s")
    nw = mesh.num_cores * mesh.num_subcores
    dest3d = dest.reshape(nw, (N // nw) // _SC_CH, _SC_CH)
    xpad = _sc_scatter(img, dest3d)
    ypad, loss = _experts(xpad, We, be, Wd, bd, blkexp, limit)
    out = _sc_gather(ypad, dest3d)
    return loss.reshape(()), out


# speedup vs baseline: 1.1376x; 1.1376x over previous
import functools

import jax
import jax.numpy as jnp
from jax import lax
from jax.experimental import pallas as pl
from jax.experimental.pallas import tpu as pltpu
from jax.experimental.pallas import tpu_sc as plsc

E = 8
D = 768
H = 128
N = 16384
BLK = 512
NB = N // BLK + E
NPAD = NB * BLK

RG = 128
RC = N // RG


def _route_body(lab_ref, dest_ref, blkexp_ref, limit_ref):
    lab = lab_ref[...]
    io0 = lax.broadcasted_iota(jnp.int32, (RC, RC), 0)
    io1 = lax.broadcasted_iota(jnp.int32, (RC, RC), 1)
    m_cols = (io0 < io1).astype(jnp.float32)
    l_rows = (io1 < io0).astype(jnp.float32)

    counts = []
    offs = []
    dest = jnp.zeros((RG, RC), dtype=jnp.int32)
    off = jnp.int32(0)
    for e in range(E):
        mask = (lab == e)
        maskf = mask.astype(jnp.float32)
        within = jnp.dot(maskf, m_cols, preferred_element_type=jnp.float32)
        rowcnt = jnp.sum(maskf, axis=1, keepdims=True)
        rowpre = jnp.dot(l_rows, rowcnt,
                         preferred_element_type=jnp.float32)
        rank = (within + rowpre).astype(jnp.int32)
        cnt = jnp.sum(mask.astype(jnp.int32))
        counts.append(cnt)
        offs.append(off)
        dest = dest + jnp.where(mask, off + rank, 0)
        padded = ((cnt + BLK - 1) // BLK) * BLK
        off = off + padded
    dest_ref[...] = dest

    brow = lax.broadcasted_iota(jnp.int32, (1, NB), 1) * BLK
    blkexp = jnp.zeros((1, NB), dtype=jnp.int32)
    limit = jnp.zeros((1, NB), dtype=jnp.int32)
    for e in range(E):
        lo = offs[e]
        hi = offs[e + 1] if e + 1 < E else off
        ind = (brow >= lo) & (brow < hi)
        blkexp = blkexp + jnp.where(ind, e, 0)
        limit = limit + jnp.where(ind, lo + counts[e], 0)
    blkexp_ref[...] = blkexp
    limit_ref[...] = limit


def _route(label2d):
    return pl.pallas_call(
        _route_body,
        out_shape=(
            jax.ShapeDtypeStruct((RG, RC), jnp.int32),
            jax.ShapeDtypeStruct((1, NB), jnp.int32),
            jax.ShapeDtypeStruct((1, NB), jnp.int32),
        ),
    )(label2d)


_SC_CH = 128


def _sc_scatter(img, dest3d):
    mesh = plsc.VectorSubcoreMesh(core_axis_name="c", subcore_axis_name="s")
    nw = mesh.num_cores * mesh.num_subcores
    tpw = N // nw
    nch = tpw // _SC_CH

    @functools.partial(
        pl.kernel,
        out_type=jax.ShapeDtypeStruct((NPAD, D), jnp.float32),
        mesh=mesh,
        scratch_types=[
            pltpu.VMEM((nch, _SC_CH), jnp.int32),
            pltpu.VMEM((_SC_CH, D), jnp.float32),
            pltpu.SemaphoreType.DMA,
        ],
    )
    def k(img_hbm, dest_hbm, xpad_hbm, idx_v, rows_v, sem):
        wid = lax.axis_index("s") * mesh.num_cores + lax.axis_index("c")
        base = wid * tpw
        pltpu.sync_copy(dest_hbm.at[wid], idx_v)
        for j in range(nch):
            pltpu.sync_copy(img_hbm.at[pl.ds(base + j * _SC_CH, _SC_CH)],
                            rows_v)
            pltpu.async_copy(rows_v, xpad_hbm.at[idx_v.at[j]], sem).wait()

    return k(img, dest3d)


def _sc_gather(ypad, dest3d):
    mesh = plsc.VectorSubcoreMesh(core_axis_name="c", subcore_axis_name="s")
    nw = mesh.num_cores * mesh.num_subcores
    tpw = N // nw
    nch = tpw // _SC_CH

    @functools.partial(
        pl.kernel,
        out_type=jax.ShapeDtypeStruct((N, D), jnp.float32),
        mesh=mesh,
        scratch_types=[
            pltpu.VMEM((nch, _SC_CH), jnp.int32),
            pltpu.VMEM((_SC_CH, D), jnp.float32),
            pltpu.SemaphoreType.DMA,
        ],
    )
    def k(ypad_hbm, dest_hbm, out_hbm, idx_v, rows_v, sem):
        wid = lax.axis_index("s") * mesh.num_cores + lax.axis_index("c")
        base = wid * tpw
        pltpu.sync_copy(dest_hbm.at[wid], idx_v)
        for j in range(nch):
            pltpu.async_copy(ypad_hbm.at[idx_v.at[j]], rows_v, sem).wait()
            pltpu.sync_copy(rows_v, out_hbm.at[pl.ds(base + j * _SC_CH, _SC_CH)])

    return k(ypad, dest3d)


def _expert_body(be_idx_ref, lim_ref, x_ref, we_ref, bee_ref, wd_ref, bd_ref,
                 y_ref, loss_ref, acc_ref):
    b = pl.program_id(0)
    x = x_ref[...]
    h = jnp.dot(x.astype(jnp.bfloat16), we_ref[0].astype(jnp.bfloat16),
                preferred_element_type=jnp.float32)
    h = h + bee_ref[0]
    y = jnp.dot(h.astype(jnp.bfloat16), wd_ref[0].astype(jnp.bfloat16),
                preferred_element_type=jnp.float32)
    y = y + bd_ref[0]
    y_ref[...] = y

    limit = lim_ref[0, b]
    row = b * BLK + lax.broadcasted_iota(jnp.int32, (BLK, 1), 0)
    diff = y - x
    sq = jnp.where(row < limit, diff * diff, 0.0)

    @pl.when(b == 0)
    def _():
        acc_ref[0] = 0.0

    acc_ref[0] += jnp.sum(sq)

    @pl.when(b == NB - 1)
    def _():
        loss_ref[...] = jnp.reshape(acc_ref[0] / (N * D), (1, 1))


def _experts(xpad, We, be, Wd, bd, blkexp, limit):
    grid_spec = pltpu.PrefetchScalarGridSpec(
        num_scalar_prefetch=2,
        grid=(NB,),
        in_specs=[
            pl.BlockSpec((BLK, D), lambda b, bexp, lim: (b, 0)),
            pl.BlockSpec((1, D, H), lambda b, bexp, lim: (bexp[0, b], 0, 0)),
            pl.BlockSpec((1, 1, H), lambda b, bexp, lim: (bexp[0, b], 0, 0)),
            pl.BlockSpec((1, H, D), lambda b, bexp, lim: (bexp[0, b], 0, 0)),
            pl.BlockSpec((1, 1, D), lambda b, bexp, lim: (bexp[0, b], 0, 0)),
        ],
        out_specs=[
            pl.BlockSpec((BLK, D), lambda b, bexp, lim: (b, 0)),
            pl.BlockSpec((1, 1), lambda b, bexp, lim: (0, 0)),
        ],
        scratch_shapes=[pltpu.SMEM((1,), jnp.float32)],
    )
    return pl.pallas_call(
        _expert_body,
        grid_spec=grid_spec,
        out_shape=(
            jax.ShapeDtypeStruct((NPAD, D), jnp.float32),
            jax.ShapeDtypeStruct((1, 1), jnp.float32),
        ),
    )(blkexp, limit, xpad, We, be.reshape(E, 1, H), Wd, bd.reshape(E, 1, D))


def kernel(img, label, We, be, Wd, bd):
    label2d = label.astype(jnp.int32).reshape(RG, RC)
    dest, blkexp, limit = _route(label2d)
    mesh = plsc.VectorSubcoreMesh(core_axis_name="c", subcore_axis_name="s")
    nw = mesh.num_cores * mesh.num_subcores
    dest3d = dest.reshape(nw, (N // nw) // _SC_CH, _SC_CH)
    xpad = _sc_scatter(img, dest3d)
    ypad, loss = _experts(xpad, We, be, Wd, bd, blkexp, limit)
    out = _sc_gather(ypad, dest3d)
    return loss.reshape(()), out


# speedup vs baseline: 1.1719x; 1.0301x over previous
import functools

import jax
import jax.numpy as jnp
from jax import lax
from jax.experimental import pallas as pl
from jax.experimental.pallas import tpu as pltpu
from jax.experimental.pallas import tpu_sc as plsc

E = 8
D = 768
H = 128
N = 16384
BLK = 512
NB = N // BLK + E
NPAD = NB * BLK

RG = 128
RC = N // RG


def _route_body(lab_ref, dest_ref, blkexp_ref, limit_ref):
    lab = lab_ref[...]
    io0 = lax.broadcasted_iota(jnp.int32, (RC, RC), 0)
    io1 = lax.broadcasted_iota(jnp.int32, (RC, RC), 1)
    m_cols = (io0 < io1).astype(jnp.float32)
    l_rows = (io1 < io0).astype(jnp.float32)

    counts = []
    offs = []
    dest = jnp.zeros((RG, RC), dtype=jnp.int32)
    off = jnp.int32(0)
    for e in range(E):
        mask = (lab == e)
        maskf = mask.astype(jnp.float32)
        within = jnp.dot(maskf, m_cols, preferred_element_type=jnp.float32)
        rowcnt = jnp.sum(maskf, axis=1, keepdims=True)
        rowpre = jnp.dot(l_rows, rowcnt,
                         preferred_element_type=jnp.float32)
        rank = (within + rowpre).astype(jnp.int32)
        cnt = jnp.sum(mask.astype(jnp.int32))
        counts.append(cnt)
        offs.append(off)
        dest = dest + jnp.where(mask, off + rank, 0)
        padded = ((cnt + BLK - 1) // BLK) * BLK
        off = off + padded
    dest_ref[...] = dest

    brow = lax.broadcasted_iota(jnp.int32, (1, NB), 1) * BLK
    blkexp = jnp.zeros((1, NB), dtype=jnp.int32)
    limit = jnp.zeros((1, NB), dtype=jnp.int32)
    for e in range(E):
        lo = offs[e]
        hi = offs[e + 1] if e + 1 < E else off
        ind = (brow >= lo) & (brow < hi)
        blkexp = blkexp + jnp.where(ind, e, 0)
        limit = limit + jnp.where(ind, lo + counts[e], 0)
    blkexp_ref[...] = blkexp
    limit_ref[...] = limit


def _route(label2d):
    return pl.pallas_call(
        _route_body,
        out_shape=(
            jax.ShapeDtypeStruct((RG, RC), jnp.int32),
            jax.ShapeDtypeStruct((1, NB), jnp.int32),
            jax.ShapeDtypeStruct((1, NB), jnp.int32),
        ),
    )(label2d)


_SC_CH = 128


def _sc_scatter(img, dest3d):
    mesh = plsc.VectorSubcoreMesh(core_axis_name="c", subcore_axis_name="s")
    nw = mesh.num_cores * mesh.num_subcores
    tpw = N // nw
    nch = tpw // _SC_CH

    @functools.partial(
        pl.kernel,
        out_type=jax.ShapeDtypeStruct((NPAD, D), jnp.float32),
        mesh=mesh,
        scratch_types=[
            pltpu.VMEM((nch, _SC_CH), jnp.int32),
            pltpu.VMEM((_SC_CH, D), jnp.float32),
            pltpu.SemaphoreType.DMA,
        ],
    )
    def k(img_hbm, dest_hbm, xpad_hbm, idx_v, rows_v, sem):
        wid = lax.axis_index("s") * mesh.num_cores + lax.axis_index("c")
        base = wid * tpw
        pltpu.sync_copy(dest_hbm.at[wid], idx_v)
        for j in range(nch):
            pltpu.sync_copy(img_hbm.at[pl.ds(base + j * _SC_CH, _SC_CH)],
                            rows_v)
            pltpu.async_copy(rows_v, xpad_hbm.at[idx_v.at[j]], sem).wait()

    return k(img, dest3d)


def _sc_gather(ypad, dest3d):
    mesh = plsc.VectorSubcoreMesh(core_axis_name="c", subcore_axis_name="s")
    nw = mesh.num_cores * mesh.num_subcores
    tpw = N // nw
    nch = tpw // _SC_CH

    @functools.partial(
        pl.kernel,
        out_type=jax.ShapeDtypeStruct((N, D), jnp.float32),
        mesh=mesh,
        scratch_types=[
            pltpu.VMEM((nch, _SC_CH), jnp.int32),
            pltpu.VMEM((_SC_CH, D), jnp.float32),
            pltpu.SemaphoreType.DMA,
        ],
    )
    def k(ypad_hbm, dest_hbm, out_hbm, idx_v, rows_v, sem):
        wid = lax.axis_index("s") * mesh.num_cores + lax.axis_index("c")
        base = wid * tpw
        pltpu.sync_copy(dest_hbm.at[wid], idx_v)
        for j in range(nch):
            pltpu.async_copy(ypad_hbm.at[idx_v.at[j]], rows_v, sem).wait()
            pltpu.sync_copy(rows_v, out_hbm.at[pl.ds(base + j * _SC_CH, _SC_CH)])

    return k(ypad, dest3d)


def _expert_body(be_idx_ref, lim_ref, x_ref, we_ref, bee_ref, wd_ref, bd_ref,
                 y_ref, loss_ref, acc_ref):
    b = pl.program_id(0)
    x = x_ref[...]
    h = jnp.dot(x.astype(jnp.bfloat16), we_ref[0].astype(jnp.bfloat16),
                preferred_element_type=jnp.float32)
    h = h + bee_ref[0]
    y = jnp.dot(h.astype(jnp.bfloat16), wd_ref[0].astype(jnp.bfloat16),
                preferred_element_type=jnp.float32)
    y = y + bd_ref[0]
    y_ref[...] = y

    limit = lim_ref[0, b]
    row = b * BLK + lax.broadcasted_iota(jnp.int32, (1, BLK), 1)
    maskv = (row < limit).astype(jnp.float32)
    diff = y - x
    rowsum = jnp.dot(maskv, diff * diff,
                     preferred_element_type=jnp.float32)

    @pl.when(b == 0)
    def _():
        acc_ref[...] = jnp.zeros((1, D), jnp.float32)

    acc_ref[...] += rowsum

    @pl.when(b == NB - 1)
    def _():
        loss_ref[...] = jnp.reshape(jnp.sum(acc_ref[...]) / (N * D), (1, 1))


def _experts(xpad, We, be, Wd, bd, blkexp, limit):
    grid_spec = pltpu.PrefetchScalarGridSpec(
        num_scalar_prefetch=2,
        grid=(NB,),
        in_specs=[
            pl.BlockSpec((BLK, D), lambda b, bexp, lim: (b, 0)),
            pl.BlockSpec((1, D, H), lambda b, bexp, lim: (bexp[0, b], 0, 0)),
            pl.BlockSpec((1, 1, H), lambda b, bexp, lim: (bexp[0, b], 0, 0)),
            pl.BlockSpec((1, H, D), lambda b, bexp, lim: (bexp[0, b], 0, 0)),
            pl.BlockSpec((1, 1, D), lambda b, bexp, lim: (bexp[0, b], 0, 0)),
        ],
        out_specs=[
            pl.BlockSpec((BLK, D), lambda b, bexp, lim: (b, 0)),
            pl.BlockSpec((1, 1), lambda b, bexp, lim: (0, 0)),
        ],
        scratch_shapes=[pltpu.VMEM((1, D), jnp.float32)],
    )
    return pl.pallas_call(
        _expert_body,
        grid_spec=grid_spec,
        out_shape=(
            jax.ShapeDtypeStruct((NPAD, D), jnp.float32),
            jax.ShapeDtypeStruct((1, 1), jnp.float32),
        ),
    )(blkexp, limit, xpad, We, be.reshape(E, 1, H), Wd, bd.reshape(E, 1, D))


def kernel(img, label, We, be, Wd, bd):
    label2d = label.astype(jnp.int32).reshape(RG, RC)
    dest, blkexp, limit = _route(label2d)
    mesh = plsc.VectorSubcoreMesh(core_axis_name="c", subcore_axis_name="s")
    nw = mesh.num_cores * mesh.num_subcores
    dest3d = dest.reshape(nw, (N // nw) // _SC_CH, _SC_CH)
    xpad = _sc_scatter(img, dest3d)
    ypad, loss = _experts(xpad, We, be, Wd, bd, blkexp, limit)
    out = _sc_gather(ypad, dest3d)
    return loss.reshape(()), out


# speedup vs baseline: 1.6447x; 1.4035x over previous
import jax
import jax.numpy as jnp
from jax import lax
from jax.experimental import pallas as pl
from jax.experimental.pallas import tpu as pltpu

E = 8
D = 768
H = 128
N = 16384
BLK = 256
NBLK = N // BLK
HA = E * H
HAUG = HA + 128


def _body(lab_ref, x_ref, weall_ref, beall_ref, wdaug_ref, y_ref, loss_ref,
          acc_ref):
    b = pl.program_id(0)
    x = x_ref[...]
    h_all = jnp.dot(x.astype(jnp.bfloat16), weall_ref[...],
                    preferred_element_type=jnp.float32)

    lab = lab_ref[0]
    pieces = []
    for e in range(E):
        he = h_all[:, e * H:(e + 1) * H] + beall_ref[:, e * H:(e + 1) * H]
        pieces.append(jnp.where(lab == e, he, 0.0).astype(jnp.bfloat16))
    onehot = (lab == lax.broadcasted_iota(jnp.int32, (BLK, 128), 1)
              ).astype(jnp.bfloat16)
    hcat = jnp.concatenate(pieces + [onehot], axis=1)

    y = jnp.dot(hcat, wdaug_ref[...],
                preferred_element_type=jnp.float32)
    y_ref[...] = y

    diff = y - x
    ones = jnp.ones((1, BLK), jnp.float32)
    rowsum = jnp.dot(ones, diff * diff,
                     preferred_element_type=jnp.float32)

    @pl.when(b == 0)
    def _():
        acc_ref[...] = jnp.zeros((1, D), jnp.float32)

    acc_ref[...] += rowsum

    @pl.when(b == NBLK - 1)
    def _():
        loss_ref[...] = jnp.reshape(jnp.sum(acc_ref[...]) / (N * D), (1, 1))


def kernel(img, label, We, be, Wd, bd):
    lab3d = label.astype(jnp.int32).reshape(NBLK, BLK, 1)
    we_all = jnp.transpose(We, (1, 0, 2)).reshape(D, HA).astype(jnp.bfloat16)
    be_all = be.reshape(1, HA)
    wd_aug = jnp.concatenate(
        [Wd.reshape(HA, D), bd, jnp.zeros((HAUG - HA - E, D), jnp.float32)],
        axis=0).astype(jnp.bfloat16)

    grid_spec = pltpu.PrefetchScalarGridSpec(
        num_scalar_prefetch=0,
        grid=(NBLK,),
        in_specs=[
            pl.BlockSpec((1, BLK, 1), lambda b: (b, 0, 0)),
            pl.BlockSpec((BLK, D), lambda b: (b, 0)),
            pl.BlockSpec((D, HA), lambda b: (0, 0)),
            pl.BlockSpec((1, HA), lambda b: (0, 0)),
            pl.BlockSpec((HAUG, D), lambda b: (0, 0)),
        ],
        out_specs=[
            pl.BlockSpec((BLK, D), lambda b: (b, 0)),
            pl.BlockSpec((1, 1), lambda b: (0, 0)),
        ],
        scratch_shapes=[pltpu.VMEM((1, D), jnp.float32)],
    )
    out, loss = pl.pallas_call(
        _body,
        grid_spec=grid_spec,
        out_shape=(
            jax.ShapeDtypeStruct((N, D), jnp.float32),
            jax.ShapeDtypeStruct((1, 1), jnp.float32),
        ),
    )(lab3d, img, we_all, be_all, wd_aug)
    return loss.reshape(()), out


# speedup vs baseline: 1.6980x; 1.0324x over previous
import jax
import jax.numpy as jnp
from jax import lax
from jax.experimental import pallas as pl
from jax.experimental.pallas import tpu as pltpu

E = 8
D = 768
H = 128
N = 16384
BLK = 256
NBLK = N // BLK
HA = E * H


def _body(lab_ref, x_ref, weall_ref, lanee_ref, wdall_ref, bfull_ref,
          y_ref, loss_ref, acc_ref):
    b = pl.program_id(0)
    x = x_ref[...]
    h_all = jnp.dot(x.astype(jnp.bfloat16), weall_ref[...],
                    preferred_element_type=jnp.float32)

    lab = lab_ref[0]
    maskb = (lanee_ref[...] == lab).astype(jnp.bfloat16)
    hm = h_all.astype(jnp.bfloat16) * maskb
    onehot = (lab == lax.broadcasted_iota(jnp.int32, (BLK, H), 1)
              ).astype(jnp.bfloat16)

    y = (jnp.dot(hm, wdall_ref[...], preferred_element_type=jnp.float32)
         + jnp.dot(onehot, bfull_ref[...],
                   preferred_element_type=jnp.float32))
    y_ref[...] = y

    diff = y - x
    ones = jnp.ones((1, BLK), jnp.float32)
    rowsum = jnp.dot(ones, diff * diff,
                     preferred_element_type=jnp.float32)

    @pl.when(b == 0)
    def _():
        acc_ref[...] = jnp.zeros((1, D), jnp.float32)

    acc_ref[...] += rowsum

    @pl.when(b == NBLK - 1)
    def _():
        loss_ref[...] = jnp.reshape(jnp.sum(acc_ref[...]) / (N * D), (1, 1))


def kernel(img, label, We, be, Wd, bd):
    lab3d = label.astype(jnp.int32).reshape(NBLK, BLK, 1)
    we_all = jnp.transpose(We, (1, 0, 2)).reshape(D, HA).astype(jnp.bfloat16)
    wd_all = Wd.reshape(HA, D).astype(jnp.bfloat16)
    lane_e = (jnp.arange(HA, dtype=jnp.int32) // H).reshape(1, HA)
    bfull = jnp.concatenate(
        [jnp.einsum("eh,ehd->ed", be, Wd) + bd,
         jnp.zeros((H - E, D), jnp.float32)], axis=0).astype(jnp.bfloat16)

    grid_spec = pltpu.PrefetchScalarGridSpec(
        num_scalar_prefetch=0,
        grid=(NBLK,),
        in_specs=[
            pl.BlockSpec((1, BLK, 1), lambda b: (b, 0, 0)),
            pl.BlockSpec((BLK, D), lambda b: (b, 0)),
            pl.BlockSpec((D, HA), lambda b: (0, 0)),
            pl.BlockSpec((1, HA), lambda b: (0, 0)),
            pl.BlockSpec((HA, D), lambda b: (0, 0)),
            pl.BlockSpec((H, D), lambda b: (0, 0)),
        ],
        out_specs=[
            pl.BlockSpec((BLK, D), lambda b: (b, 0)),
            pl.BlockSpec((1, 1), lambda b: (0, 0)),
        ],
        scratch_shapes=[pltpu.VMEM((1, D), jnp.float32)],
    )
    out, loss = pl.pallas_call(
        _body,
        grid_spec=grid_spec,
        out_shape=(
            jax.ShapeDtypeStruct((N, D), jnp.float32),
            jax.ShapeDtypeStruct((1, 1), jnp.float32),
        ),
    )(lab3d, img, we_all, lane_e, wd_all, bfull)
    return loss.reshape(()), out


# speedup vs baseline: 2.0185x; 1.1887x over previous
import jax
import jax.numpy as jnp
from jax import lax
from jax.experimental import pallas as pl
from jax.experimental.pallas import tpu as pltpu

E = 8
D = 768
H = 128
N = 16384
BLK = 512
NBLK = N // BLK
HA = E * H


def _body(lab_ref, x_ref, weall_ref, lanee_ref, wdall_ref, bfull_ref,
          y_ref, loss_ref, acc_ref):
    b = pl.program_id(0)
    x = x_ref[...]
    h_all = jnp.dot(x.astype(jnp.bfloat16), weall_ref[...],
                    preferred_element_type=jnp.float32)

    lab = lab_ref[0]
    maskb = (lanee_ref[...] == lab).astype(jnp.bfloat16)
    hm = h_all.astype(jnp.bfloat16) * maskb
    onehot = (lab == lax.broadcasted_iota(jnp.int32, (BLK, H), 1)
              ).astype(jnp.bfloat16)

    y = (jnp.dot(hm, wdall_ref[...], preferred_element_type=jnp.float32)
         + jnp.dot(onehot, bfull_ref[...],
                   preferred_element_type=jnp.float32))
    y_ref[...] = y

    diff = y - x
    ones = jnp.ones((1, BLK), jnp.float32)
    rowsum = jnp.dot(ones, diff * diff,
                     preferred_element_type=jnp.float32)

    @pl.when(b == 0)
    def _():
        acc_ref[...] = jnp.zeros((1, D), jnp.float32)

    acc_ref[...] += rowsum

    @pl.when(b == NBLK - 1)
    def _():
        loss_ref[...] = jnp.reshape(jnp.sum(acc_ref[...]) / (N * D), (1, 1))


def kernel(img, label, We, be, Wd, bd):
    lab3d = label.astype(jnp.int32).reshape(NBLK, BLK, 1)
    we_all = jnp.transpose(We, (1, 0, 2)).reshape(D, HA).astype(jnp.bfloat16)
    wd_all = Wd.reshape(HA, D).astype(jnp.bfloat16)
    lane_e = (jnp.arange(HA, dtype=jnp.int32) // H).reshape(1, HA)
    bfull = jnp.concatenate(
        [jnp.einsum("eh,ehd->ed", be, Wd) + bd,
         jnp.zeros((H - E, D), jnp.float32)], axis=0).astype(jnp.bfloat16)

    grid_spec = pltpu.PrefetchScalarGridSpec(
        num_scalar_prefetch=0,
        grid=(NBLK,),
        in_specs=[
            pl.BlockSpec((1, BLK, 1), lambda b: (b, 0, 0)),
            pl.BlockSpec((BLK, D), lambda b: (b, 0)),
            pl.BlockSpec((D, HA), lambda b: (0, 0)),
            pl.BlockSpec((1, HA), lambda b: (0, 0)),
            pl.BlockSpec((HA, D), lambda b: (0, 0)),
            pl.BlockSpec((H, D), lambda b: (0, 0)),
        ],
        out_specs=[
            pl.BlockSpec((BLK, D), lambda b: (b, 0)),
            pl.BlockSpec((1, 1), lambda b: (0, 0)),
        ],
        scratch_shapes=[pltpu.VMEM((1, D), jnp.float32)],
    )
    out, loss = pl.pallas_call(
        _body,
        grid_spec=grid_spec,
        out_shape=(
            jax.ShapeDtypeStruct((N, D), jnp.float32),
            jax.ShapeDtypeStruct((1, 1), jnp.float32),
        ),
    )(lab3d, img, we_all, lane_e, wd_all, bfull)
    return loss.reshape(()), out


# speedup vs baseline: 2.1440x; 1.0622x over previous
import jax
import jax.numpy as jnp
from jax import lax
from jax.experimental import pallas as pl
from jax.experimental.pallas import tpu as pltpu

E = 8
D = 768
H = 128
N = 16384
BLK = 1024
NBLK = N // BLK
HA = E * H


def _body(lab_ref, x_ref, weall_ref, lanee_ref, wdall_ref, bfull_ref,
          y_ref, loss_ref, acc_ref):
    b = pl.program_id(0)
    x = x_ref[...]
    h_all = jnp.dot(x.astype(jnp.bfloat16), weall_ref[...],
                    preferred_element_type=jnp.float32)

    lab = lab_ref[0]
    maskb = (lanee_ref[...] == lab).astype(jnp.bfloat16)
    hm = h_all.astype(jnp.bfloat16) * maskb
    onehot = (lab == lax.broadcasted_iota(jnp.int32, (BLK, H), 1)
              ).astype(jnp.bfloat16)

    y = (jnp.dot(hm, wdall_ref[...], preferred_element_type=jnp.float32)
         + jnp.dot(onehot, bfull_ref[...],
                   preferred_element_type=jnp.float32))
    y_ref[...] = y

    diff = y - x
    ones = jnp.ones((1, BLK), jnp.float32)
    rowsum = jnp.dot(ones, diff * diff,
                     preferred_element_type=jnp.float32)

    @pl.when(b == 0)
    def _():
        acc_ref[...] = jnp.zeros((1, D), jnp.float32)

    acc_ref[...] += rowsum

    @pl.when(b == NBLK - 1)
    def _():
        loss_ref[...] = jnp.reshape(jnp.sum(acc_ref[...]) / (N * D), (1, 1))


def kernel(img, label, We, be, Wd, bd):
    lab3d = label.astype(jnp.int32).reshape(NBLK, BLK, 1)
    we_all = jnp.transpose(We, (1, 0, 2)).reshape(D, HA).astype(jnp.bfloat16)
    wd_all = Wd.reshape(HA, D).astype(jnp.bfloat16)
    lane_e = (jnp.arange(HA, dtype=jnp.int32) // H).reshape(1, HA)
    bfull = jnp.concatenate(
        [jnp.einsum("eh,ehd->ed", be, Wd) + bd,
         jnp.zeros((H - E, D), jnp.float32)], axis=0).astype(jnp.bfloat16)

    grid_spec = pltpu.PrefetchScalarGridSpec(
        num_scalar_prefetch=0,
        grid=(NBLK,),
        in_specs=[
            pl.BlockSpec((1, BLK, 1), lambda b: (b, 0, 0)),
            pl.BlockSpec((BLK, D), lambda b: (b, 0)),
            pl.BlockSpec((D, HA), lambda b: (0, 0)),
            pl.BlockSpec((1, HA), lambda b: (0, 0)),
            pl.BlockSpec((HA, D), lambda b: (0, 0)),
            pl.BlockSpec((H, D), lambda b: (0, 0)),
        ],
        out_specs=[
            pl.BlockSpec((BLK, D), lambda b: (b, 0)),
            pl.BlockSpec((1, 1), lambda b: (0, 0)),
        ],
        scratch_shapes=[pltpu.VMEM((1, D), jnp.float32)],
    )
    out, loss = pl.pallas_call(
        _body,
        grid_spec=grid_spec,
        out_shape=(
            jax.ShapeDtypeStruct((N, D), jnp.float32),
            jax.ShapeDtypeStruct((1, 1), jnp.float32),
        ),
    )(lab3d, img, we_all, lane_e, wd_all, bfull)
    return loss.reshape(()), out
